# baseline (speedup 1.0000x reference)
from functools import partial

import jax
import jax.numpy as jnp
from jax.experimental import pallas as pl
from jax.experimental.pallas import tpu as pltpu


def _mm_kernel(n_batches, nk, tk, fin, x_hbm, l_ref, o_ref, xb_ref, stage_ref, sem):
    j = pl.program_id(0)

    @pl.when(j == 0)
    def _load_x():
        for k in range(min(2, nk)):
            pltpu.make_async_copy(
                x_hbm.at[:, pl.ds(k * tk, tk), :], stage_ref.at[k], sem.at[k]
            ).start()
        for k in range(nk):
            slot = k % 2
            pltpu.make_async_copy(
                x_hbm.at[:, pl.ds(k * tk, tk), :], stage_ref.at[slot], sem.at[slot]
            ).wait()
            for n in range(n_batches):
                xb_ref[pl.ds(k * tk, tk), n * fin : (n + 1) * fin] = stage_ref[
                    slot, n
                ].astype(jnp.bfloat16)
            if k + 2 < nk:
                pltpu.make_async_copy(
                    x_hbm.at[:, pl.ds((k + 2) * tk, tk), :],
                    stage_ref.at[slot],
                    sem.at[slot],
                ).start()

    l = l_ref[...].astype(jnp.bfloat16)
    y2 = jnp.dot(l, xb_ref[...], preferred_element_type=jnp.float32)
    for n in range(n_batches):
        o_ref[n] = y2[:, n * fin : (n + 1) * fin]


def kernel(x, L):
    N, M, Fin = x.shape
    Mp = L.shape[0]
    TM = 512
    TK = 512
    out = pl.pallas_call(
        partial(_mm_kernel, N, M // TK, TK, Fin),
        grid=(Mp // TM,),
        in_specs=[
            pl.BlockSpec(memory_space=pl.MemorySpace.ANY),
            pl.BlockSpec((TM, M), lambda i: (i, 0)),
        ],
        out_specs=pl.BlockSpec((N, TM, Fin), lambda i: (0, i, 0)),
        out_shape=jax.ShapeDtypeStruct((N, Mp, Fin), jnp.float32),
        scratch_shapes=[
            pltpu.VMEM((M, N * Fin), jnp.bfloat16),
            pltpu.VMEM((2, N, TK, Fin), jnp.float32),
            pltpu.SemaphoreType.DMA((2,)),
        ],
    )(x, L)
    return out


# speedup vs baseline: 1.0689x; 1.0689x over previous
from functools import partial

import jax
import jax.numpy as jnp
from jax.experimental import pallas as pl
from jax.experimental.pallas import tpu as pltpu


def _mm_kernel(n_batches, nk, tk, fin, x_hbm, l_ref, o_ref, xb_ref, stage_ref, sem):
    j = pl.program_id(0)
    l = l_ref[...].astype(jnp.bfloat16)

    @pl.when(j == 0)
    def _first_step():
        for k in range(min(2, nk)):
            pltpu.make_async_copy(
                x_hbm.at[:, pl.ds(k * tk, tk), :], stage_ref.at[k], sem.at[k]
            ).start()
        for k in range(nk):
            slot = k % 3
            pltpu.make_async_copy(
                x_hbm.at[:, pl.ds(k * tk, tk), :], stage_ref.at[slot], sem.at[slot]
            ).wait()
            if k + 2 < nk:
                nslot = (k + 2) % 3
                pltpu.make_async_copy(
                    x_hbm.at[:, pl.ds((k + 2) * tk, tk), :],
                    stage_ref.at[nslot],
                    sem.at[nslot],
                ).start()
            for n in range(n_batches):
                xb_ref[pl.ds(k * tk, tk), n * fin : (n + 1) * fin] = stage_ref[
                    slot, n
                ].astype(jnp.bfloat16)
            dk = jnp.dot(
                l[:, k * tk : (k + 1) * tk],
                xb_ref[pl.ds(k * tk, tk), :],
                preferred_element_type=jnp.float32,
            )
            for n in range(n_batches):
                if k == 0:
                    o_ref[n] = dk[:, n * fin : (n + 1) * fin]
                else:
                    o_ref[n] += dk[:, n * fin : (n + 1) * fin]

    @pl.when(j > 0)
    def _rest():
        y2 = jnp.dot(l, xb_ref[...], preferred_element_type=jnp.float32)
        for n in range(n_batches):
            o_ref[n] = y2[:, n * fin : (n + 1) * fin]


def kernel(x, L):
    N, M, Fin = x.shape
    Mp = L.shape[0]
    TM = 512
    TK = 512
    out = pl.pallas_call(
        partial(_mm_kernel, N, M // TK, TK, Fin),
        grid=(Mp // TM,),
        in_specs=[
            pl.BlockSpec(memory_space=pl.MemorySpace.ANY),
            pl.BlockSpec((TM, M), lambda i: (i, 0)),
        ],
        out_specs=pl.BlockSpec((N, TM, Fin), lambda i: (0, i, 0)),
        out_shape=jax.ShapeDtypeStruct((N, Mp, Fin), jnp.float32),
        scratch_shapes=[
            pltpu.VMEM((M, N * Fin), jnp.bfloat16),
            pltpu.VMEM((3, N, TK, Fin), jnp.float32),
            pltpu.SemaphoreType.DMA((3,)),
        ],
    )(x, L)
    return out
